# baseline (speedup 1.0000x reference)
import functools

import jax
import jax.numpy as jnp
from jax import lax
from jax.experimental import pallas as pl
from jax.experimental.pallas import tpu as pltpu
from jax.experimental.pallas import tpu_sc as plsc


def sc_unpool(updates, mask):
    B, H, W, C = updates.shape
    Ho, Wo = 2 * H, 2 * W
    WoC = Wo * C
    NW = 32
    ROWS = (B * H) // NW
    K6 = C // 16

    mesh = plsc.VectorSubcoreMesh(core_axis_name="c", subcore_axis_name="s")

    @functools.partial(
        pl.kernel,
        out_type=jax.ShapeDtypeStruct((B, Ho, Wo, C), jnp.float32),
        mesh=mesh,
        scratch_types=[
            pltpu.VMEM((W, C), jnp.float32),
            pltpu.VMEM((W, C), jnp.int32),
            pltpu.VMEM((2, Wo, C), jnp.float32),
        ],
        compiler_params=pltpu.CompilerParams(use_tc_tiling_on_sc=True),
    )
    def k(upd_hbm, mask_hbm, out_hbm, ubuf, mbuf, obuf):
        wid = lax.axis_index("s") * 2 + lax.axis_index("c")
        row0 = wid * ROWS
        lane = lax.iota(jnp.int32, 16)

        def row_body(t, carry):
            i = row0 + t
            b = i // H
            h = i % H
            pltpu.sync_copy(upd_hbm.at[b, h], ubuf)
            pltpu.sync_copy(mask_hbm.at[b, h], mbuf)
            rowbase = (b * Ho + 2 * h) * WoC

            def w_body(w, carry2):
                for k6 in range(K6):
                    c0 = k6 * 16
                    u = ubuf[w, pl.ds(c0, 16)]
                    m = mbuf[w, pl.ds(c0, 16)]
                    d = m - (rowbase + 2 * w * C + c0 + lane)
                    for dy in range(2):
                        for dx in range(2):
                            v = jnp.where(d == dy * WoC + dx * C, u, 0.0)
                            obuf[dy, 2 * w + dx, pl.ds(c0, 16)] = v
                return carry2

            lax.fori_loop(0, W, w_body, 0)
            pltpu.sync_copy(obuf, out_hbm.at[b, pl.ds(2 * h, 2)])
            return carry

        lax.fori_loop(0, ROWS, row_body, 0)

    return k(updates, mask)


def kernel(updates, mask):
    return sc_unpool(updates, mask.astype(jnp.int32))


# speedup vs baseline: 1.1565x; 1.1565x over previous
import functools

import jax
import jax.numpy as jnp
from jax import lax
from jax.experimental import pallas as pl
from jax.experimental.pallas import tpu as pltpu
from jax.experimental.pallas import tpu_sc as plsc


def sc_unpool(updates, mask):
    B, H, W, C = updates.shape
    Ho, Wo = 2 * H, 2 * W
    WoC = Wo * C
    NW = 32
    ROWS = (B * H) // NW
    K6 = C // 16

    mesh = plsc.VectorSubcoreMesh(core_axis_name="c", subcore_axis_name="s")

    @functools.partial(
        pl.kernel,
        out_type=jax.ShapeDtypeStruct((B, Ho, Wo, C), jnp.float32),
        mesh=mesh,
        scratch_types=[
            pltpu.VMEM((2, W, C), jnp.float32),
            pltpu.VMEM((2, W, C), jnp.int32),
            pltpu.VMEM((2, Wo, C), jnp.float32),
            pltpu.SemaphoreType.DMA,
            pltpu.SemaphoreType.DMA,
            pltpu.SemaphoreType.DMA,
            pltpu.SemaphoreType.DMA,
            pltpu.SemaphoreType.DMA,
        ],
        compiler_params=pltpu.CompilerParams(use_tc_tiling_on_sc=True),
    )
    def k(upd_hbm, mask_hbm, out_hbm, ubuf, mbuf, obuf,
          su0, su1, sm0, sm1, so):
        su = (su0, su1)
        sm = (sm0, sm1)
        wid = lax.axis_index("s") * 2 + lax.axis_index("c")
        row0 = wid * ROWS
        lane = lax.iota(jnp.int32, 16)

        def bh(t):
            i = row0 + t
            return i // H, i % H

        for ph in range(2):
            b, h = bh(ph)
            pltpu.async_copy(upd_hbm.at[b, h], ubuf.at[ph], su[ph])
            pltpu.async_copy(mask_hbm.at[b, h], mbuf.at[ph], sm[ph])

        def block(tt, carry):
            for ph in range(2):
                t = 2 * tt + ph
                b, h = bh(t)
                pltpu.make_async_copy(upd_hbm.at[b, h], ubuf.at[ph], su[ph]).wait()
                pltpu.make_async_copy(mask_hbm.at[b, h], mbuf.at[ph], sm[ph]).wait()

                if ph == 0:
                    @pl.when(tt >= 1)
                    def _():
                        pltpu.make_async_copy(
                            obuf, out_hbm.at[b, pl.ds(2 * h, 2)], so
                        ).wait()
                else:
                    pltpu.make_async_copy(
                        obuf, out_hbm.at[b, pl.ds(2 * h, 2)], so
                    ).wait()

                rowbase = (b * Ho + 2 * h) * WoC

                def w_body(w, carry2):
                    for k6 in range(K6):
                        c0 = k6 * 16
                        u = ubuf[ph, w, pl.ds(c0, 16)]
                        m = mbuf[ph, w, pl.ds(c0, 16)]
                        d = m - (rowbase + 2 * w * C + c0 + lane)
                        for dy in range(2):
                            for dx in range(2):
                                v = jnp.where(d == dy * WoC + dx * C, u, 0.0)
                                obuf[dy, 2 * w + dx, pl.ds(c0, 16)] = v
                    return carry2

                lax.fori_loop(0, W, w_body, 0)

                pltpu.async_copy(obuf, out_hbm.at[b, pl.ds(2 * h, 2)], so)

                @pl.when(t + 2 < ROWS)
                def _():
                    b2, h2 = bh(t + 2)
                    pltpu.async_copy(upd_hbm.at[b2, h2], ubuf.at[ph], su[ph])
                    pltpu.async_copy(mask_hbm.at[b2, h2], mbuf.at[ph], sm[ph])
            return carry

        lax.fori_loop(0, ROWS // 2, block, 0)

        b, h = bh(ROWS - 1)
        pltpu.make_async_copy(obuf, out_hbm.at[b, pl.ds(2 * h, 2)], so).wait()

    return k(updates, mask)


def kernel(updates, mask):
    return sc_unpool(updates, mask.astype(jnp.int32))
